# baseline (speedup 1.0000x reference)
import functools

import jax
import jax.numpy as jnp
from jax import lax
from jax.experimental import pallas as pl
from jax.experimental.pallas import tpu as pltpu

K_NN = 20
BB = 32


def _patch_kernel(ptsw_ref, xbn_ref, ones_ref, w1t_ref, b1_ref, w2t_ref,
                  b2_ref, w3_ref, b3_ref, out_ref, cur_ref, m_ref,
                  *, n, k, bb):
    f32 = jnp.float32
    ptsw = ptsw_ref[...]
    w = bb * n

    xxr = jnp.sum(ptsw * ptsw, axis=0, keepdims=True)
    diag = (lax.broadcasted_iota(jnp.int32, (n, n), 0)
            == lax.broadcasted_iota(jnp.int32, (n, n), 1))
    for p in range(bb):
        lo, hi = p * n, (p + 1) * n
        xbn = xbn_ref[p]
        G = jnp.dot(xbn, ptsw[:, lo:hi], preferred_element_type=f32)
        xxc = jnp.sum(xbn * xbn, axis=1, keepdims=True)
        cur_ref[:, lo:hi] = jnp.where(
            diag, -jnp.inf, 2.0 * G - xxr[:, lo:hi] - xxc)

    qT = jnp.dot(w1t_ref[...], ptsw, preferred_element_type=f32)
    m_ref[...] = qT

    def body2(t, carry):
        cur = cur_ref[...]
        max1 = jnp.max(cur, axis=0, keepdims=True)
        is1 = cur >= max1
        eff = jnp.where(is1, -jnp.inf, cur)
        max2 = jnp.max(eff, axis=0, keepdims=True)
        is2 = eff >= max2
        hf1 = is1.astype(f32)
        hf2 = is2.astype(f32)
        mm = m_ref[...]
        for p in range(bb):
            lo, hi = p * n, (p + 1) * n
            q1 = jnp.dot(qT[:, lo:hi], hf1[:, lo:hi],
                         preferred_element_type=f32)
            q2 = jnp.dot(qT[:, lo:hi], hf2[:, lo:hi],
                         preferred_element_type=f32)
            m_ref[:, lo:hi] = jnp.maximum(mm[:, lo:hi], jnp.maximum(q1, q2))
        cur_ref[...] = jnp.where(is2, -jnp.inf, eff)
        return carry

    lax.fori_loop(0, (k - 1) // 2, body2, 0, unroll=3)

    cur = cur_ref[...]
    colmax = jnp.max(cur, axis=0, keepdims=True)
    hf = (cur >= colmax).astype(f32)
    mm = m_ref[...]
    for p in range(bb):
        lo, hi = p * n, (p + 1) * n
        qsel = jnp.dot(qT[:, lo:hi], hf[:, lo:hi],
                       preferred_element_type=f32)
        m_ref[:, lo:hi] = jnp.maximum(mm[:, lo:hi], qsel)

    f1 = jnp.maximum(m_ref[...] - qT + b1_ref[...], 0.0)
    f2 = jnp.dot(w2t_ref[...], f1, preferred_element_type=f32) + b2_ref[...]
    f2 = jnp.maximum(f2, 0.0)
    logits = jnp.sum(f2 * w3_ref[...], axis=0, keepdims=True) + b3_ref[0, 0]
    wgt = 1.0 / (1.0 + jnp.exp(-logits))

    x = ptsw[0:1, :]
    y = ptsw[1:2, :]
    z = ptsw[2:3, :]
    xy = x * y
    x2 = x * x
    y2 = y * y
    onesr = jnp.ones((1, w), f32)
    ar = [x, y, x2, y2, xy, onesr]
    prods = []
    for a in range(6):
        for b in range(a, 6):
            prods.append(ar[a] * ar[b])
    for a in range(6):
        prods.append(ar[a] * z)
    nprod = len(prods)
    rows = []
    rows.append(jnp.concatenate(prods, axis=0))
    rows.append(jnp.concatenate([pr * wgt for pr in prods], axis=0))
    rows.append(jnp.concatenate([(wgt > 0.001).astype(f32),
                                 jnp.abs(x), jnp.abs(y)], axis=0))
    stat = jnp.concatenate(rows, axis=0)
    sums = jnp.dot(stat, ones_ref[...], preferred_element_type=f32)

    valid = sums[2 * nprod:2 * nprod + 1, :]
    use_w = valid > 18.0
    h = (sums[2 * nprod + 1:2 * nprod + 2, :]
         + sums[2 * nprod + 2:2 * nprod + 3, :]) / (2.0 * n)
    h = jnp.where(jnp.abs(h) < 1e-4, 0.1, h)
    hinv = 1.0 / h

    s = [hinv, hinv, hinv * hinv, hinv * hinv, hinv * hinv,
         jnp.ones_like(hinv)]
    M = [[None] * 6 for _ in range(6)]
    rhs = [None] * 6
    idx = 0
    for a in range(6):
        for b in range(a, 6):
            v = jnp.where(use_w, sums[nprod + idx:nprod + idx + 1, :],
                          sums[idx:idx + 1, :]) * (s[a] * s[b])
            M[a][b] = v + 1e-6 if a == b else v
            M[b][a] = M[a][b]
            idx += 1
    for a in range(6):
        j = 21 + a
        rhs[a] = jnp.where(use_w, sums[nprod + j:nprod + j + 1, :],
                           sums[j:j + 1, :]) * s[a]

    for kk in range(6):
        inv = 1.0 / M[kk][kk]
        for r in range(kk + 1, 6):
            f = M[r][kk] * inv
            for c in range(kk + 1, 6):
                M[r][c] = M[r][c] - f * M[kk][c]
            rhs[r] = rhs[r] - f * rhs[kk]
    beta = [None] * 6
    for r in range(5, -1, -1):
        sv = rhs[r]
        for c in range(r + 1, 6):
            sv = sv - M[r][c] * beta[c]
        beta[r] = sv / M[r][r]

    nx = -beta[0] * hinv
    ny = -beta[1] * hinv
    rs = lax.rsqrt(nx * nx + ny * ny + 1.0)
    out_ref[0] = jnp.concatenate([nx * rs, ny * rs, rs], axis=0)


def kernel(points, W1, b1, W2, b2, W3, b3):
    B, _, N = points.shape
    ptsw = jnp.transpose(points, (1, 0, 2)).reshape(3, B * N)
    xbn = jnp.transpose(points, (0, 2, 1))
    ones_blk = (jnp.arange(BB * N)[:, None] // N
                == jnp.arange(BB)[None, :]).astype(jnp.float32)
    w1t = W1.T
    b1c = b1.reshape(64, 1)
    w2t = W2.T
    b2c = b2.reshape(128, 1)
    w3c = W3.reshape(128, 1)
    b3c = b3.reshape(1, 1)

    body = functools.partial(_patch_kernel, n=N, k=K_NN, bb=BB)
    out = pl.pallas_call(
        body,
        grid=(B // BB,),
        in_specs=[
            pl.BlockSpec((3, BB * N), lambda i: (0, i)),
            pl.BlockSpec((BB, N, 3), lambda i: (i, 0, 0)),
            pl.BlockSpec((BB * N, BB), lambda i: (0, 0)),
            pl.BlockSpec((64, 3), lambda i: (0, 0)),
            pl.BlockSpec((64, 1), lambda i: (0, 0)),
            pl.BlockSpec((128, 64), lambda i: (0, 0)),
            pl.BlockSpec((128, 1), lambda i: (0, 0)),
            pl.BlockSpec((128, 1), lambda i: (0, 0)),
            pl.BlockSpec((1, 1), lambda i: (0, 0)),
        ],
        out_specs=pl.BlockSpec((1, 3, BB), lambda i: (i, 0, 0)),
        out_shape=jax.ShapeDtypeStruct((B // BB, 3, BB), jnp.float32),
        scratch_shapes=[
            pltpu.VMEM((N, BB * N), jnp.float32),
            pltpu.VMEM((64, BB * N), jnp.float32),
        ],
        compiler_params=pltpu.CompilerParams(
            dimension_semantics=("arbitrary",)),
    )(ptsw, xbn, ones_blk, w1t, b1c, w2t, b2c, w3c, b3c)
    return out.transpose(0, 2, 1).reshape(B, 3)


# speedup vs baseline: 1.1074x; 1.1074x over previous
import functools

import jax
import jax.numpy as jnp
from jax import lax
from jax.experimental import pallas as pl
from jax.experimental.pallas import tpu as pltpu

K_NN = 20
BB = 32


def _patch_kernel(ptsw_ref, xbn_ref, ones_ref, w1t_ref, b1_ref, w2t_ref,
                  b2_ref, w3_ref, b3_ref, out_ref, cur_ref, m_ref,
                  *, n, k, bb):
    f32 = jnp.float32
    ptsw = ptsw_ref[...]
    w = bb * n

    xxr = jnp.sum(ptsw * ptsw, axis=0, keepdims=True)
    diag = (lax.broadcasted_iota(jnp.int32, (n, n), 0)
            == lax.broadcasted_iota(jnp.int32, (n, n), 1))
    for p in range(bb):
        lo, hi = p * n, (p + 1) * n
        xbn = xbn_ref[p]
        G = jnp.dot(xbn, ptsw[:, lo:hi], preferred_element_type=f32)
        xxc = jnp.sum(xbn * xbn, axis=1, keepdims=True)
        cur_ref[:, lo:hi] = jnp.where(
            diag, -jnp.inf, 2.0 * G - xxr[:, lo:hi] - xxc)

    qT = jnp.dot(w1t_ref[...], ptsw, preferred_element_type=f32)
    m_ref[...] = qT

    colmax0 = jnp.max(cur_ref[...], axis=0, keepdims=True)

    def body(t, colmax):
        cur = cur_ref[...]
        ismax = cur >= colmax
        curnew = jnp.where(ismax, -jnp.inf, cur)
        cur_ref[...] = curnew
        nextmax = jnp.max(curnew, axis=0, keepdims=True)
        hf = ismax.astype(f32)
        mm = m_ref[...]
        for p in range(bb):
            lo, hi = p * n, (p + 1) * n
            qsel = jnp.dot(qT[:, lo:hi], hf[:, lo:hi],
                           preferred_element_type=f32)
            m_ref[:, lo:hi] = jnp.maximum(mm[:, lo:hi], qsel)
        return nextmax

    lax.fori_loop(0, k - 1, body, colmax0, unroll=4)

    f1 = jnp.maximum(m_ref[...] - qT + b1_ref[...], 0.0)
    f2 = jnp.dot(w2t_ref[...], f1, preferred_element_type=f32) + b2_ref[...]
    f2 = jnp.maximum(f2, 0.0)
    logits = jnp.sum(f2 * w3_ref[...], axis=0, keepdims=True) + b3_ref[0, 0]
    wgt = 1.0 / (1.0 + jnp.exp(-logits))

    x = ptsw[0:1, :]
    y = ptsw[1:2, :]
    z = ptsw[2:3, :]
    xy = x * y
    x2 = x * x
    y2 = y * y
    onesr = jnp.ones((1, w), f32)
    ar = [x, y, x2, y2, xy, onesr]
    prods = []
    for a in range(6):
        for b in range(a, 6):
            prods.append(ar[a] * ar[b])
    for a in range(6):
        prods.append(ar[a] * z)
    nprod = len(prods)
    rows = []
    rows.append(jnp.concatenate(prods, axis=0))
    rows.append(jnp.concatenate([pr * wgt for pr in prods], axis=0))
    rows.append(jnp.concatenate([(wgt > 0.001).astype(f32),
                                 jnp.abs(x), jnp.abs(y)], axis=0))
    stat = jnp.concatenate(rows, axis=0)
    sums = jnp.dot(stat, ones_ref[...], preferred_element_type=f32)

    valid = sums[2 * nprod:2 * nprod + 1, :]
    use_w = valid > 18.0
    h = (sums[2 * nprod + 1:2 * nprod + 2, :]
         + sums[2 * nprod + 2:2 * nprod + 3, :]) / (2.0 * n)
    h = jnp.where(jnp.abs(h) < 1e-4, 0.1, h)
    hinv = 1.0 / h

    s = [hinv, hinv, hinv * hinv, hinv * hinv, hinv * hinv,
         jnp.ones_like(hinv)]
    M = [[None] * 6 for _ in range(6)]
    rhs = [None] * 6
    idx = 0
    for a in range(6):
        for b in range(a, 6):
            v = jnp.where(use_w, sums[nprod + idx:nprod + idx + 1, :],
                          sums[idx:idx + 1, :]) * (s[a] * s[b])
            M[a][b] = v + 1e-6 if a == b else v
            M[b][a] = M[a][b]
            idx += 1
    for a in range(6):
        j = 21 + a
        rhs[a] = jnp.where(use_w, sums[nprod + j:nprod + j + 1, :],
                           sums[j:j + 1, :]) * s[a]

    for kk in range(6):
        inv = 1.0 / M[kk][kk]
        for r in range(kk + 1, 6):
            f = M[r][kk] * inv
            for c in range(kk + 1, 6):
                M[r][c] = M[r][c] - f * M[kk][c]
            rhs[r] = rhs[r] - f * rhs[kk]
    beta = [None] * 6
    for r in range(5, -1, -1):
        sv = rhs[r]
        for c in range(r + 1, 6):
            sv = sv - M[r][c] * beta[c]
        beta[r] = sv / M[r][r]

    nx = -beta[0] * hinv
    ny = -beta[1] * hinv
    rs = lax.rsqrt(nx * nx + ny * ny + 1.0)
    out_ref[0] = jnp.concatenate([nx * rs, ny * rs, rs], axis=0)


def kernel(points, W1, b1, W2, b2, W3, b3):
    B, _, N = points.shape
    ptsw = jnp.transpose(points, (1, 0, 2)).reshape(3, B * N)
    xbn = jnp.transpose(points, (0, 2, 1))
    ones_blk = (jnp.arange(BB * N)[:, None] // N
                == jnp.arange(BB)[None, :]).astype(jnp.float32)
    w1t = W1.T
    b1c = b1.reshape(64, 1)
    w2t = W2.T
    b2c = b2.reshape(128, 1)
    w3c = W3.reshape(128, 1)
    b3c = b3.reshape(1, 1)

    body = functools.partial(_patch_kernel, n=N, k=K_NN, bb=BB)
    out = pl.pallas_call(
        body,
        grid=(B // BB,),
        in_specs=[
            pl.BlockSpec((3, BB * N), lambda i: (0, i)),
            pl.BlockSpec((BB, N, 3), lambda i: (i, 0, 0)),
            pl.BlockSpec((BB * N, BB), lambda i: (0, 0)),
            pl.BlockSpec((64, 3), lambda i: (0, 0)),
            pl.BlockSpec((64, 1), lambda i: (0, 0)),
            pl.BlockSpec((128, 64), lambda i: (0, 0)),
            pl.BlockSpec((128, 1), lambda i: (0, 0)),
            pl.BlockSpec((128, 1), lambda i: (0, 0)),
            pl.BlockSpec((1, 1), lambda i: (0, 0)),
        ],
        out_specs=pl.BlockSpec((1, 3, BB), lambda i: (i, 0, 0)),
        out_shape=jax.ShapeDtypeStruct((B // BB, 3, BB), jnp.float32),
        scratch_shapes=[
            pltpu.VMEM((N, BB * N), jnp.float32),
            pltpu.VMEM((64, BB * N), jnp.float32),
        ],
        compiler_params=pltpu.CompilerParams(
            dimension_semantics=("arbitrary",)),
    )(ptsw, xbn, ones_blk, w1t, b1c, w2t, b2c, w3c, b3c)
    return out.transpose(0, 2, 1).reshape(B, 3)


# speedup vs baseline: 1.1151x; 1.0069x over previous
import functools

import jax
import jax.numpy as jnp
from jax import lax
from jax.experimental import pallas as pl
from jax.experimental.pallas import tpu as pltpu

K_NN = 20
BB = 32


def _patch_kernel(ptsw_ref, xbn_ref, ones_ref, w1t_ref, b1_ref, w2t_ref,
                  b2_ref, w3_ref, b3_ref, out_ref, cur_ref, m_ref,
                  *, n, k, bb):
    f32 = jnp.float32
    ptsw = ptsw_ref[...]
    w = bb * n

    xxr = jnp.sum(ptsw * ptsw, axis=0, keepdims=True)
    diag = (lax.broadcasted_iota(jnp.int32, (n, n), 0)
            == lax.broadcasted_iota(jnp.int32, (n, n), 1))
    for p in range(bb):
        lo, hi = p * n, (p + 1) * n
        xbn = xbn_ref[p]
        G = jnp.dot(xbn, ptsw[:, lo:hi], preferred_element_type=f32)
        xxc = jnp.sum(xbn * xbn, axis=1, keepdims=True)
        cur_ref[:, lo:hi] = jnp.where(
            diag, -jnp.inf, 2.0 * G - xxr[:, lo:hi] - xxc)

    qT = jnp.dot(w1t_ref[...], ptsw, preferred_element_type=f32)
    m_ref[...] = qT

    colmax0 = jnp.max(cur_ref[...], axis=0, keepdims=True)

    def body(t, colmax):
        cur = cur_ref[...]
        ismax = cur >= colmax
        curnew = jnp.where(ismax, -jnp.inf, cur)
        cur_ref[...] = curnew
        nextmax = jnp.max(curnew, axis=0, keepdims=True)
        hf = ismax.astype(f32)
        mm = m_ref[...]
        for p in range(bb):
            lo, hi = p * n, (p + 1) * n
            qsel = jnp.dot(qT[:, lo:hi], hf[:, lo:hi],
                           preferred_element_type=f32)
            m_ref[:, lo:hi] = jnp.maximum(mm[:, lo:hi], qsel)
        return nextmax

    lax.fori_loop(0, k - 1, body, colmax0, unroll=8)

    f1 = jnp.maximum(m_ref[...] - qT + b1_ref[...], 0.0)
    f2 = jnp.dot(w2t_ref[...], f1, preferred_element_type=f32) + b2_ref[...]
    f2 = jnp.maximum(f2, 0.0)
    logits = jnp.sum(f2 * w3_ref[...], axis=0, keepdims=True) + b3_ref[0, 0]
    wgt = 1.0 / (1.0 + jnp.exp(-logits))

    x = ptsw[0:1, :]
    y = ptsw[1:2, :]
    z = ptsw[2:3, :]
    xy = x * y
    x2 = x * x
    y2 = y * y
    onesr = jnp.ones((1, w), f32)
    ar = [x, y, x2, y2, xy, onesr]
    prods = []
    for a in range(6):
        for b in range(a, 6):
            prods.append(ar[a] * ar[b])
    for a in range(6):
        prods.append(ar[a] * z)
    nprod = len(prods)
    rows = []
    rows.append(jnp.concatenate(prods, axis=0))
    rows.append(jnp.concatenate([pr * wgt for pr in prods], axis=0))
    rows.append(jnp.concatenate([(wgt > 0.001).astype(f32),
                                 jnp.abs(x), jnp.abs(y)], axis=0))
    stat = jnp.concatenate(rows, axis=0)
    sums = jnp.dot(stat, ones_ref[...], preferred_element_type=f32)

    valid = sums[2 * nprod:2 * nprod + 1, :]
    use_w = valid > 18.0
    h = (sums[2 * nprod + 1:2 * nprod + 2, :]
         + sums[2 * nprod + 2:2 * nprod + 3, :]) / (2.0 * n)
    h = jnp.where(jnp.abs(h) < 1e-4, 0.1, h)
    hinv = 1.0 / h

    s = [hinv, hinv, hinv * hinv, hinv * hinv, hinv * hinv,
         jnp.ones_like(hinv)]
    M = [[None] * 6 for _ in range(6)]
    rhs = [None] * 6
    idx = 0
    for a in range(6):
        for b in range(a, 6):
            v = jnp.where(use_w, sums[nprod + idx:nprod + idx + 1, :],
                          sums[idx:idx + 1, :]) * (s[a] * s[b])
            M[a][b] = v + 1e-6 if a == b else v
            M[b][a] = M[a][b]
            idx += 1
    for a in range(6):
        j = 21 + a
        rhs[a] = jnp.where(use_w, sums[nprod + j:nprod + j + 1, :],
                           sums[j:j + 1, :]) * s[a]

    for kk in range(6):
        inv = 1.0 / M[kk][kk]
        for r in range(kk + 1, 6):
            f = M[r][kk] * inv
            for c in range(kk + 1, 6):
                M[r][c] = M[r][c] - f * M[kk][c]
            rhs[r] = rhs[r] - f * rhs[kk]
    beta = [None] * 6
    for r in range(5, -1, -1):
        sv = rhs[r]
        for c in range(r + 1, 6):
            sv = sv - M[r][c] * beta[c]
        beta[r] = sv / M[r][r]

    nx = -beta[0] * hinv
    ny = -beta[1] * hinv
    rs = lax.rsqrt(nx * nx + ny * ny + 1.0)
    out_ref[0] = jnp.concatenate([nx * rs, ny * rs, rs], axis=0)


def kernel(points, W1, b1, W2, b2, W3, b3):
    B, _, N = points.shape
    ptsw = jnp.transpose(points, (1, 0, 2)).reshape(3, B * N)
    xbn = jnp.transpose(points, (0, 2, 1))
    ones_blk = (jnp.arange(BB * N)[:, None] // N
                == jnp.arange(BB)[None, :]).astype(jnp.float32)
    w1t = W1.T
    b1c = b1.reshape(64, 1)
    w2t = W2.T
    b2c = b2.reshape(128, 1)
    w3c = W3.reshape(128, 1)
    b3c = b3.reshape(1, 1)

    body = functools.partial(_patch_kernel, n=N, k=K_NN, bb=BB)
    out = pl.pallas_call(
        body,
        grid=(B // BB,),
        in_specs=[
            pl.BlockSpec((3, BB * N), lambda i: (0, i)),
            pl.BlockSpec((BB, N, 3), lambda i: (i, 0, 0)),
            pl.BlockSpec((BB * N, BB), lambda i: (0, 0)),
            pl.BlockSpec((64, 3), lambda i: (0, 0)),
            pl.BlockSpec((64, 1), lambda i: (0, 0)),
            pl.BlockSpec((128, 64), lambda i: (0, 0)),
            pl.BlockSpec((128, 1), lambda i: (0, 0)),
            pl.BlockSpec((128, 1), lambda i: (0, 0)),
            pl.BlockSpec((1, 1), lambda i: (0, 0)),
        ],
        out_specs=pl.BlockSpec((1, 3, BB), lambda i: (i, 0, 0)),
        out_shape=jax.ShapeDtypeStruct((B // BB, 3, BB), jnp.float32),
        scratch_shapes=[
            pltpu.VMEM((N, BB * N), jnp.float32),
            pltpu.VMEM((64, BB * N), jnp.float32),
        ],
        compiler_params=pltpu.CompilerParams(
            dimension_semantics=("arbitrary",)),
    )(ptsw, xbn, ones_blk, w1t, b1c, w2t, b2c, w3c, b3c)
    return out.transpose(0, 2, 1).reshape(B, 3)
